# Initial kernel scaffold; baseline (speedup 1.0000x reference)
import jax
import jax.numpy as jnp
from jax.experimental import pallas as pl


def kernel(t, h, edge_index, W0, b0, W1, b1, W2, b2, ln0_g, ln0_b, ln1_g, ln1_b, ln2_g, ln2_b, gate_W, gate_b, res_w):
    raise NotImplementedError("write your pallas kernel here")


# speedup vs baseline: 8.1636x; 8.1636x over previous
import functools

import jax
import jax.numpy as jnp
from jax import lax
from jax.experimental import pallas as pl
from jax.experimental.pallas import tpu as pltpu
from jax.experimental.pallas import tpu_sc as plsc

_NC = 2
_NS = 16
_NW = _NC * _NS

_CHUNK = 80
_ZROWS = 128


def _sc_degree(dst, n_nodes):
    e = dst.shape[0]
    epw = e // _NW
    nch = epw // _CHUNK
    zlen = 640
    zstride = 624
    mesh = plsc.VectorSubcoreMesh(
        core_axis_name="c", subcore_axis_name="s",
        num_cores=_NC, num_subcores=_NS)

    @functools.partial(
        pl.kernel,
        out_type=jax.ShapeDtypeStruct((_NC * n_nodes,), jnp.float32),
        mesh=mesh,
        scratch_types=[
            pltpu.VMEM((_CHUNK,), jnp.int32),
            pltpu.VMEM((_CHUNK,), jnp.float32),
            pltpu.VMEM((zlen,), jnp.float32),
            pltpu.VMEM_SHARED((n_nodes,), jnp.float32),
        ],
    )
    def k(dst_hbm, ones_hbm, zeros_hbm, out_hbm, didx, ones_v, zb, acc):
        cid = lax.axis_index("c")
        sid = lax.axis_index("s")
        wid = cid * _NS + sid
        pltpu.sync_copy(ones_hbm, ones_v)
        pltpu.sync_copy(zeros_hbm, zb)
        pltpu.sync_copy(zb, acc.at[pl.ds(sid * zstride, zlen)])
        plsc.subcore_barrier()

        def body(g, _):
            off = wid * epw + g * _CHUNK
            pltpu.sync_copy(dst_hbm.at[pl.ds(off, _CHUNK)], didx)
            pltpu.sync_copy(ones_v, acc.at[didx], add=True)
            return 0

        lax.fori_loop(0, nch, body, 0)
        plsc.subcore_barrier()
        pltpu.sync_copy(acc.at[pl.ds(sid * zstride, zlen)], zb)
        pltpu.sync_copy(zb, out_hbm.at[pl.ds(cid * n_nodes + sid * zstride,
                                             zlen)])

    ones = jnp.ones((_CHUNK,), jnp.float32)
    zeros = jnp.zeros((zlen,), jnp.float32)
    return k(dst, ones, zeros).reshape(_NC, n_nodes)


def _sc_edge_sum(u, src, dst):
    n, d = u.shape
    e = src.shape[0]
    epw = e // _NW
    nch = epw // _CHUNK
    rstride = 624
    nz = 5
    mesh = plsc.VectorSubcoreMesh(
        core_axis_name="c", subcore_axis_name="s",
        num_cores=_NC, num_subcores=_NS)

    @functools.partial(
        pl.kernel,
        out_type=jax.ShapeDtypeStruct((_NC, n, d), jnp.float32),
        mesh=mesh,
        scratch_types=[
            pltpu.VMEM((_CHUNK,), jnp.int32),
            pltpu.VMEM((_CHUNK,), jnp.int32),
            pltpu.VMEM((_CHUNK, d), jnp.float32),
            pltpu.VMEM((_ZROWS, d), jnp.float32),
            pltpu.VMEM_SHARED((n, d), jnp.float32),
            pltpu.SemaphoreType.DMA,
        ],
    )
    def k(u_hbm, src_hbm, dst_hbm, zeros_hbm, out_hbm, sidx, didx, rows, zb,
          acc, sem):
        cid = lax.axis_index("c")
        sid = lax.axis_index("s")
        wid = cid * _NS + sid
        pltpu.sync_copy(zeros_hbm, zb)
        for j in range(nz):
            pltpu.sync_copy(zb, acc.at[pl.ds(sid * rstride + j * _ZROWS,
                                             _ZROWS)])
        plsc.subcore_barrier()

        def body(g, _):
            off = wid * epw + g * _CHUNK
            pltpu.sync_copy(src_hbm.at[pl.ds(off, _CHUNK)], sidx)
            pltpu.sync_copy(dst_hbm.at[pl.ds(off, _CHUNK)], didx)
            pltpu.async_copy(u_hbm.at[sidx], rows, sem).wait()
            pltpu.sync_copy(rows, acc.at[didx], add=True)
            return 0

        lax.fori_loop(0, nch, body, 0)
        plsc.subcore_barrier()
        for j in range(nz):
            r0 = sid * rstride + j * _ZROWS
            pltpu.sync_copy(acc.at[pl.ds(r0, _ZROWS)], zb)
            pltpu.sync_copy(zb, out_hbm.at[cid, pl.ds(r0, _ZROWS)])

    zeros = jnp.zeros((_ZROWS, d), jnp.float32)
    return k(u, src, dst, zeros)


_BR = 1000


def _tc_pre(h, w0, degp3):
    n, d = h.shape

    def body(h_ref, w_ref, dp_ref, u_ref, dis_ref):
        deg = dp_ref[0] + dp_ref[1]
        dis = jnp.where(deg > 0, lax.rsqrt(deg), 0.0)
        dis_ref[...] = dis
        u_ref[...] = jnp.dot(h_ref[...], w_ref[...],
                             preferred_element_type=jnp.float32) * dis

    return pl.pallas_call(
        body,
        grid=(n // _BR,),
        in_specs=[
            pl.BlockSpec((_BR, d), lambda i: (i, 0)),
            pl.BlockSpec((d, d), lambda i: (0, 0)),
            pl.BlockSpec((2, _BR, 1), lambda i: (0, i, 0)),
        ],
        out_specs=[
            pl.BlockSpec((_BR, d), lambda i: (i, 0)),
            pl.BlockSpec((_BR, 1), lambda i: (i, 0)),
        ],
        out_shape=[
            jax.ShapeDtypeStruct((n, d), jnp.float32),
            jax.ShapeDtypeStruct((n, 1), jnp.float32),
        ],
    )(h, w0, degp3)


def _ln(x, g, b):
    mu = jnp.mean(x, axis=-1, keepdims=True)
    xc = x - mu
    var = jnp.mean(xc * xc, axis=-1, keepdims=True)
    return xc * lax.rsqrt(var + 1e-5) * g + b


def _tc_mid0(p, dis, b0, g0, be0, w1):
    n, d = p.shape[1], p.shape[2]

    def body(p_ref, dis_ref, b_ref, g_ref, be_ref, w_ref, h_ref, u_ref):
        dis = dis_ref[...]
        s = (p_ref[0] + p_ref[1]) * dis + b_ref[...]
        hn = _ln(s, g_ref[...], be_ref[...])
        h_ref[...] = hn
        u_ref[...] = jnp.dot(hn, w_ref[...],
                             preferred_element_type=jnp.float32) * dis

    row = lambda i: (i, 0)
    fix = lambda i: (0, 0)
    return pl.pallas_call(
        body,
        grid=(n // _BR,),
        in_specs=[
            pl.BlockSpec((2, _BR, d), lambda i: (0, i, 0)),
            pl.BlockSpec((_BR, 1), row),
            pl.BlockSpec((1, d), fix),
            pl.BlockSpec((1, d), fix),
            pl.BlockSpec((1, d), fix),
            pl.BlockSpec((d, d), fix),
        ],
        out_specs=[pl.BlockSpec((_BR, d), row), pl.BlockSpec((_BR, d), row)],
        out_shape=[
            jax.ShapeDtypeStruct((n, d), jnp.float32),
            jax.ShapeDtypeStruct((n, d), jnp.float32),
        ],
    )(p, dis, b0, g0, be0, w1)


def _tc_mid1(p, dis, b1, g1, be1, hprev, ga, gb, gbias, w2):
    n, d = p.shape[1], p.shape[2]

    def body(p_ref, dis_ref, b_ref, g_ref, be_ref, hp_ref, ga_ref, gb_ref,
             gbias_ref, w_ref, h_ref, u_ref):
        dis = dis_ref[...]
        hp = hp_ref[...]
        s = (p_ref[0] + p_ref[1]) * dis + b_ref[...]
        hn = _ln(s, g_ref[...], be_ref[...])
        z = (jnp.dot(hp, ga_ref[...], preferred_element_type=jnp.float32)
             + jnp.dot(hn, gb_ref[...], preferred_element_type=jnp.float32)
             + gbias_ref[...])
        gate = jax.nn.sigmoid(z)
        hg = gate * hn + (1.0 - gate) * hp
        h_ref[...] = hg
        u_ref[...] = jnp.dot(hg, w_ref[...],
                             preferred_element_type=jnp.float32) * dis

    row = lambda i: (i, 0)
    fix = lambda i: (0, 0)
    return pl.pallas_call(
        body,
        grid=(n // _BR,),
        in_specs=[
            pl.BlockSpec((2, _BR, d), lambda i: (0, i, 0)),
            pl.BlockSpec((_BR, 1), row),
            pl.BlockSpec((1, d), fix),
            pl.BlockSpec((1, d), fix),
            pl.BlockSpec((1, d), fix),
            pl.BlockSpec((_BR, d), row),
            pl.BlockSpec((d, d), fix),
            pl.BlockSpec((d, d), fix),
            pl.BlockSpec((1, d), fix),
            pl.BlockSpec((d, d), fix),
        ],
        out_specs=[pl.BlockSpec((_BR, d), row), pl.BlockSpec((_BR, d), row)],
        out_shape=[
            jax.ShapeDtypeStruct((n, d), jnp.float32),
            jax.ShapeDtypeStruct((n, d), jnp.float32),
        ],
    )(p, dis, b1, g1, be1, hprev, ga, gb, gbias, w2)


def _tc_post(p, dis, b2, g2, be2, hprev, ga, gb, gbias, h_orig, res_w):
    n, d = p.shape[1], p.shape[2]

    def body(p_ref, dis_ref, b_ref, g_ref, be_ref, hp_ref, ga_ref, gb_ref,
             gbias_ref, ho_ref, rw_ref, dh_ref):
        hp = hp_ref[...]
        s = (p_ref[0] + p_ref[1]) * dis_ref[...] + b_ref[...]
        hn = _ln(s, g_ref[...], be_ref[...])
        z = (jnp.dot(hp, ga_ref[...], preferred_element_type=jnp.float32)
             + jnp.dot(hn, gb_ref[...], preferred_element_type=jnp.float32)
             + gbias_ref[...])
        gate = jax.nn.sigmoid(z)
        hg = gate * hn + (1.0 - gate) * hp
        dh_ref[...] = jnp.tanh(hg) + rw_ref[...] * ho_ref[...]

    row = lambda i: (i, 0)
    fix = lambda i: (0, 0)
    return pl.pallas_call(
        body,
        grid=(n // _BR,),
        in_specs=[
            pl.BlockSpec((2, _BR, d), lambda i: (0, i, 0)),
            pl.BlockSpec((_BR, 1), row),
            pl.BlockSpec((1, d), fix),
            pl.BlockSpec((1, d), fix),
            pl.BlockSpec((1, d), fix),
            pl.BlockSpec((_BR, d), row),
            pl.BlockSpec((d, d), fix),
            pl.BlockSpec((d, d), fix),
            pl.BlockSpec((1, d), fix),
            pl.BlockSpec((_BR, d), row),
            pl.BlockSpec((1, 1), fix),
        ],
        out_specs=pl.BlockSpec((_BR, d), row),
        out_shape=jax.ShapeDtypeStruct((n, d), jnp.float32),
    )(p, dis, b2, g2, be2, hprev, ga, gb, gbias, h_orig, res_w)


def kernel(t, h, edge_index, W0, b0, W1, b1, W2, b2, ln0_g, ln0_b, ln1_g,
           ln1_b, ln2_g, ln2_b, gate_W, gate_b, res_w):
    n, d = h.shape
    ei = edge_index.astype(jnp.int32)
    src = ei[0]
    dst = ei[1]

    degp = _sc_degree(dst, n)
    degp3 = degp.reshape(_NC, n, 1)

    b0r = b0.reshape(1, d)
    b1r = b1.reshape(1, d)
    b2r = b2.reshape(1, d)
    g0 = ln0_g.reshape(1, d)
    be0 = ln0_b.reshape(1, d)
    g1 = ln1_g.reshape(1, d)
    be1 = ln1_b.reshape(1, d)
    g2 = ln2_g.reshape(1, d)
    be2 = ln2_b.reshape(1, d)
    ga = gate_W[:d]
    gb = gate_W[d:]
    gbias = gate_b.reshape(1, d)
    rw = res_w.reshape(1, 1)

    u0, dis = _tc_pre(h, W0, degp3)
    p0 = _sc_edge_sum(u0, src, dst)
    h1, u1 = _tc_mid0(p0, dis, b0r, g0, be0, W1)
    p1 = _sc_edge_sum(u1, src, dst)
    h2, u2 = _tc_mid1(p1, dis, b1r, g1, be1, h1, ga, gb, gbias, W2)
    p2 = _sc_edge_sum(u2, src, dst)
    dh = _tc_post(p2, dis, b2r, g2, be2, h2, ga, gb, gbias, h, rw)
    return dh
